# baseline (speedup 1.0000x reference)
import functools

import jax
import jax.numpy as jnp
from jax import lax
from jax.experimental import pallas as pl
from jax.experimental.pallas import tpu as pltpu
from jax.experimental.pallas import tpu_sc as plsc

VOCAB = 1000000
EMB_DIM = 64
BATCH = 4096
SEQ = 200

N_IDX = BATCH * SEQ
NUM_WORKERS = 32
PER_WORKER = N_IDX // NUM_WORKERS
CHUNK = 512
STEPS = PER_WORKER // CHUNK


def _make_gather():
    mesh = plsc.VectorSubcoreMesh(core_axis_name="c", subcore_axis_name="s")

    @functools.partial(
        pl.kernel,
        mesh=mesh,
        out_type=jax.ShapeDtypeStruct((N_IDX, EMB_DIM), jnp.float32),
        scratch_types=[
            pltpu.VMEM((CHUNK,), jnp.int32),
            pltpu.VMEM((CHUNK, EMB_DIM), jnp.float32),
            pltpu.SemaphoreType.DMA,
        ],
        compiler_params=pltpu.CompilerParams(use_tc_tiling_on_sc=False),
    )
    def gather_kernel(idx_hbm, table_hbm, out_hbm, idx_v, rows_v, sem):
        wid = lax.axis_index("s") * 2 + lax.axis_index("c")
        wbase = wid * PER_WORKER

        def body(t, _):
            base = wbase + t * CHUNK
            pltpu.sync_copy(idx_hbm.at[pl.ds(base, CHUNK)], idx_v)
            pltpu.async_copy(table_hbm.at[idx_v], rows_v, sem).wait()
            pltpu.sync_copy(rows_v, out_hbm.at[pl.ds(base, CHUNK)])
            return ()

        lax.fori_loop(0, STEPS, body, (), unroll=False)

    return gather_kernel


_gather = _make_gather()


def kernel(input_ids, attention_mask, emb_weight):
    flat_ids = input_ids.reshape(N_IDX)
    rows = _gather(flat_ids, emb_weight)
    return rows.reshape(BATCH, SEQ, EMB_DIM), attention_mask


# speedup vs baseline: 1.0454x; 1.0454x over previous
import functools

import jax
import jax.numpy as jnp
from jax import lax
from jax.experimental import pallas as pl
from jax.experimental.pallas import tpu as pltpu
from jax.experimental.pallas import tpu_sc as plsc

VOCAB = 1000000
EMB_DIM = 64
BATCH = 4096
SEQ = 200

N_IDX = BATCH * SEQ
NUM_WORKERS = 32
PER_WORKER = N_IDX // NUM_WORKERS
CHUNK = 512
NBUF = 2
STEPS = PER_WORKER // CHUNK
ROUNDS = STEPS // NBUF


def _make_gather():
    mesh = plsc.VectorSubcoreMesh(core_axis_name="c", subcore_axis_name="s")

    @functools.partial(
        pl.kernel,
        mesh=mesh,
        out_type=jax.ShapeDtypeStruct((N_IDX, EMB_DIM), jnp.float32),
        scratch_types=[
            pltpu.VMEM((PER_WORKER,), jnp.int32),
            pltpu.VMEM((NBUF, CHUNK, EMB_DIM), jnp.float32),
            pltpu.SemaphoreType.DMA,
            pltpu.SemaphoreType.DMA,
        ],
        compiler_params=pltpu.CompilerParams(use_tc_tiling_on_sc=False),
    )
    def gather_kernel(idx_hbm, table_hbm, out_hbm, idx_v, rows_v, sem_g, sem_w):
        wid = lax.axis_index("s") * 2 + lax.axis_index("c")
        wbase = wid * PER_WORKER

        pltpu.sync_copy(idx_hbm.at[pl.ds(wbase, PER_WORKER)], idx_v)

        def start_gather(t, b):
            pltpu.async_copy(
                table_hbm.at[idx_v.at[pl.ds(t * CHUNK, CHUNK)]],
                rows_v.at[b],
                sem_g,
            )

        def wait_gather(b):
            pltpu.make_async_copy(
                table_hbm.at[idx_v.at[pl.ds(0, CHUNK)]], rows_v.at[b], sem_g
            ).wait()

        def start_write(t, b):
            pltpu.async_copy(
                rows_v.at[b],
                out_hbm.at[pl.ds(wbase + t * CHUNK, CHUNK)],
                sem_w,
            )

        def wait_write(b):
            pltpu.make_async_copy(
                rows_v.at[b], out_hbm.at[pl.ds(wbase, CHUNK)], sem_w
            ).wait()

        for b in range(NBUF):
            start_gather(b, b)

        def round_body(k, _):
            for b in range(NBUF):
                wait_gather(b)
                start_write(k * NBUF + b, b)
            for b in range(NBUF):
                @pl.when(k < ROUNDS - 1)
                def _():
                    wait_write(b)
                    start_gather((k + 1) * NBUF + b, b)
            return ()

        lax.fori_loop(0, ROUNDS, round_body, (), unroll=False)

        for b in range(NBUF):
            wait_write(b)

    return gather_kernel


_gather = _make_gather()


def kernel(input_ids, attention_mask, emb_weight):
    flat_ids = input_ids.reshape(N_IDX)
    rows = _gather(flat_ids, emb_weight)
    return rows.reshape(BATCH, SEQ, EMB_DIM), attention_mask


# speedup vs baseline: 1.2721x; 1.2168x over previous
import functools

import jax
import jax.numpy as jnp
from jax import lax
from jax.experimental import pallas as pl
from jax.experimental.pallas import tpu as pltpu
from jax.experimental.pallas import tpu_sc as plsc

VOCAB = 1000000
EMB_DIM = 64
PAD_DIM = 128
BATCH = 4096
SEQ = 200

N_IDX = BATCH * SEQ
NUM_WORKERS = 32
PER_WORKER = N_IDX // NUM_WORKERS
CHUNK = 320
NBUF = 2
STEPS = PER_WORKER // CHUNK
ROUNDS = STEPS // NBUF


def _make_gather():
    mesh = plsc.VectorSubcoreMesh(core_axis_name="c", subcore_axis_name="s")

    @functools.partial(
        pl.kernel,
        mesh=mesh,
        out_type=jax.ShapeDtypeStruct((N_IDX, PAD_DIM), jnp.float32),
        scratch_types=[
            pltpu.VMEM((PER_WORKER,), jnp.int32),
            pltpu.VMEM((NBUF, CHUNK, PAD_DIM), jnp.float32),
            pltpu.SemaphoreType.DMA,
            pltpu.SemaphoreType.DMA,
        ],
    )
    def gather_kernel(idx_hbm, table_hbm, out_hbm, idx_v, rows_v, sem_g, sem_w):
        wid = lax.axis_index("s") * 2 + lax.axis_index("c")
        wbase = wid * PER_WORKER

        pltpu.sync_copy(idx_hbm.at[pl.ds(wbase, PER_WORKER)], idx_v)

        def start_gather(t, b):
            pltpu.async_copy(
                table_hbm.at[idx_v.at[pl.ds(t * CHUNK, CHUNK)]],
                rows_v.at[b],
                sem_g,
            )

        def wait_gather(b):
            pltpu.make_async_copy(
                table_hbm.at[idx_v.at[pl.ds(0, CHUNK)]], rows_v.at[b], sem_g
            ).wait()

        def start_write(t, b):
            pltpu.async_copy(
                rows_v.at[b],
                out_hbm.at[pl.ds(wbase + t * CHUNK, CHUNK)],
                sem_w,
            )

        def wait_write(b):
            pltpu.make_async_copy(
                rows_v.at[b], out_hbm.at[pl.ds(wbase, CHUNK)], sem_w
            ).wait()

        for b in range(NBUF):
            start_gather(b, b)

        def round_body(k, _):
            for b in range(NBUF):
                wait_gather(b)
                start_write(k * NBUF + b, b)
            for b in range(NBUF):
                @pl.when(k < ROUNDS - 1)
                def _():
                    wait_write(b)
                    start_gather((k + 1) * NBUF + b, b)
            return ()

        lax.fori_loop(0, ROUNDS, round_body, (), unroll=False)

        for b in range(NBUF):
            wait_write(b)

    return gather_kernel


_gather = _make_gather()


def kernel(input_ids, attention_mask, emb_weight):
    tab128 = jnp.pad(emb_weight, ((0, 0), (0, PAD_DIM - EMB_DIM)))
    flat_ids = input_ids.reshape(N_IDX)
    rows = _gather(flat_ids, tab128)
    out = rows.reshape(BATCH, SEQ, PAD_DIM)[:, :, :EMB_DIM]
    return out, attention_mask
